# baseline (speedup 1.0000x reference)
import functools

import jax
import jax.numpy as jnp
from jax import lax
from jax.experimental import pallas as pl
from jax.experimental.pallas import tpu as pltpu
from jax.experimental.pallas import tpu_sc as plsc

MAXLEN = 200
EMBED_DIM = 32
CLASS_NUM = 128
BATCH = 4096
VOCAB = 1000000
PKW = EMBED_DIM // 2

NUM_CORES = 2
NUM_SUBCORES = 16
NUM_WORKERS = NUM_CORES * NUM_SUBCORES

HALF = 100
SAMPLES_PER_WORKER = BATCH // NUM_WORKERS
CHUNK_S = 16
CHUNK_ROWS = CHUNK_S * MAXLEN
CHUNK_GROUPS = 2 * CHUNK_S
NUM_CHUNKS = SAMPLES_PER_WORKER // CHUNK_S
GROUPS_PER_WORKER = SAMPLES_PER_WORKER * 2

_TR_C = 32768
_TR_GRID = -(-VOCAB // _TR_C)
_TR_ROWS_PAD = _TR_GRID * _TR_C
_TR_E = _TR_C // 8
_TR_E_SH = _TR_E.bit_length() - 1


def _sc_pool_body(idx_hbm, table_hbm, out_hbm, idx_v, idx2_0, idx2_1,
                  rows_0, rows_1, pooled_v, sem0, sem1):
    wid = lax.axis_index("s") * NUM_CORES + lax.axis_index("c")
    gbase_w = wid * GROUPS_PER_WORKER

    idx2 = (idx2_0, idx2_1)
    rows = (rows_0, rows_1)
    sems = (sem0, sem1)

    zero = jnp.zeros((16,), jnp.float32)
    inv = 1.0 / MAXLEN

    def stage_and_fire(c, p):
        pltpu.sync_copy(idx_hbm.at[pl.ds(gbase_w + c * CHUNK_GROUPS,
                                         CHUNK_GROUPS)], idx_v)
        for g in range(CHUNK_GROUPS):
            for j in (0, 16, 32, 48, 64, 80, HALF - 16):
                v = idx_v[g, j:j + 16]
                t = v & (_TR_C - 1)
                idx2[p][g, j:j + 16] = ((v ^ t) + ((t & (_TR_E - 1)) << 3)
                                        + (t >> _TR_E_SH))
        for g in range(CHUNK_GROUPS):
            pltpu.async_copy(table_hbm.at[idx2[p].at[g]],
                             rows[p].at[pl.ds(g * HALF, HALF)], sems[p])

    def wait_chunk(p):
        pltpu.make_async_copy(table_hbm.at[pl.ds(0, CHUNK_ROWS)],
                              rows[p], sems[p]).wait()

    def reduce_chunk(c, p):
        for s in range(CHUNK_S):
            base = s * MAXLEN

            def rbody(r, carry, base=base, rv=rows[p]):
                accs = list(carry)
                r0 = base + r * 4
                for k in range(4):
                    u = lax.bitcast_convert_type(rv[r0 + k, 0:16],
                                                 jnp.int32)
                    a = lax.bitcast_convert_type(u << 16, jnp.float32)
                    b = lax.bitcast_convert_type(u & jnp.int32(-65536),
                                                 jnp.float32)
                    accs[k] = accs[k] + a
                    accs[4 + k] = accs[4 + k] + b
                return tuple(accs)

            accs = lax.fori_loop(0, MAXLEN // 4, rbody, (zero,) * 8)
            lo = (accs[0] + accs[1]) + (accs[2] + accs[3])
            hi = (accs[4] + accs[5]) + (accs[6] + accs[7])
            row = c * CHUNK_S + s
            pooled_v[row, 0:16] = lo * inv
            pooled_v[row, 16:32] = hi * inv

    stage_and_fire(0, 0)

    def pair_body(i, _):
        for b in (0, 1):
            c = 2 * i + b
            if b == 0:
                stage_and_fire(c + 1, 1)
            else:
                @pl.when(i < NUM_CHUNKS // 2 - 1)
                def _():
                    stage_and_fire(c + 1, 0)
            wait_chunk(b)
            reduce_chunk(c, b)
        return 0

    lax.fori_loop(0, NUM_CHUNKS // 2, pair_body, 0)
    pltpu.sync_copy(pooled_v,
                    out_hbm.at[pl.ds(wid * SAMPLES_PER_WORKER,
                                     SAMPLES_PER_WORKER)])


@functools.cache
def _build_sc_pool():
    return pl.kernel(
        _sc_pool_body,
        mesh=plsc.VectorSubcoreMesh(core_axis_name="c", subcore_axis_name="s"),
        compiler_params=pltpu.CompilerParams(use_tc_tiling_on_sc=False),
        out_type=jax.ShapeDtypeStruct((BATCH, EMBED_DIM), jnp.float32),
        scratch_types=[
            pltpu.VMEM((CHUNK_GROUPS, HALF), jnp.int32),
            pltpu.VMEM((CHUNK_GROUPS, HALF), jnp.int32),
            pltpu.VMEM((CHUNK_GROUPS, HALF), jnp.int32),
            pltpu.VMEM((CHUNK_ROWS, PKW), jnp.float32),
            pltpu.VMEM((CHUNK_ROWS, PKW), jnp.float32),
            pltpu.VMEM((SAMPLES_PER_WORKER, EMBED_DIM), jnp.float32),
            pltpu.SemaphoreType.DMA,
            pltpu.SemaphoreType.DMA,
        ],
    )


def _transpose_body(in_ref, out_ref):
    row_i = lax.broadcasted_iota(jnp.int32, (8 * EMBED_DIM, 128), 0)
    lane_i = lax.broadcasted_iota(jnp.int32, (8 * EMBED_DIM, 128), 1)
    e_i = row_i // EMBED_DIM
    d_i = row_i % EMBED_DIM
    w = lane_i - PKW * e_i
    in_seg = (w >= 0) & (w < PKW)
    sel_lo = (in_seg & (d_i == 2 * w)).astype(jnp.bfloat16)
    sel_hi = (in_seg & (d_i == 2 * w + 1)).astype(jnp.bfloat16)
    x = jnp.concatenate(
        [in_ref[:, _TR_E * e:_TR_E * (e + 1)] for e in range(8)],
        axis=0).astype(jnp.bfloat16)
    acc_lo = lax.dot_general(x, sel_lo, (((0,), (0,)), ((), ())),
                             preferred_element_type=jnp.float32)
    acc_hi = lax.dot_general(x, sel_hi, (((0,), (0,)), ((), ())),
                             preferred_element_type=jnp.float32)
    u_lo = lax.bitcast_convert_type(acc_lo.astype(jnp.bfloat16),
                                    jnp.uint16).astype(jnp.uint32)
    u_hi = lax.bitcast_convert_type(acc_hi.astype(jnp.bfloat16),
                                    jnp.uint16).astype(jnp.uint32)
    out_ref[...] = lax.bitcast_convert_type(u_lo | (u_hi << 16), jnp.float32)


def _to_rowmajor_packed(table_t):
    return pl.pallas_call(
        _transpose_body,
        grid=(_TR_GRID,),
        in_specs=[pl.BlockSpec((EMBED_DIM, _TR_C), lambda i: (0, i))],
        out_specs=pl.BlockSpec((_TR_E, 128), lambda i: (i, 0)),
        out_shape=jax.ShapeDtypeStruct((_TR_ROWS_PAD * PKW // 128, 128),
                                       jnp.float32),
    )(table_t)


def _dense_softmax_body(x_ref, w_ref, b_ref, o_ref):
    logits = jnp.dot(x_ref[...], w_ref[...],
                     preferred_element_type=jnp.float32) + b_ref[...]
    m = jnp.max(logits, axis=-1, keepdims=True)
    e = jnp.exp(logits - m)
    o_ref[...] = e / jnp.sum(e, axis=-1, keepdims=True)


_TC_BLOCK = 512


def _dense_softmax(pooled, dense_w, dense_b2d):
    return pl.pallas_call(
        _dense_softmax_body,
        grid=(BATCH // _TC_BLOCK,),
        in_specs=[
            pl.BlockSpec((_TC_BLOCK, EMBED_DIM), lambda i: (i, 0)),
            pl.BlockSpec((EMBED_DIM, CLASS_NUM), lambda i: (0, 0)),
            pl.BlockSpec((1, CLASS_NUM), lambda i: (0, 0)),
        ],
        out_specs=pl.BlockSpec((_TC_BLOCK, CLASS_NUM), lambda i: (i, 0)),
        out_shape=jax.ShapeDtypeStruct((BATCH, CLASS_NUM), jnp.float32),
    )(pooled, dense_w, dense_b2d)


def kernel(inputs, embedding_table, dense_w, dense_b):
    idx = inputs.astype(jnp.int32).reshape(BATCH * 2, HALF)
    t128 = _to_rowmajor_packed(embedding_table.T)
    tbl = t128.reshape(_TR_ROWS_PAD * PKW).reshape(_TR_ROWS_PAD, PKW)
    pooled = _build_sc_pool()(idx, tbl)
    perm = jnp.array([2 * i for i in range(PKW)]
                     + [2 * i + 1 for i in range(PKW)], dtype=jnp.int32)
    w_perm = jnp.take(dense_w, perm, axis=0)
    return _dense_softmax(pooled, w_perm,
                          dense_b.reshape(1, CLASS_NUM).astype(jnp.float32))


# speedup vs baseline: 1.0219x; 1.0219x over previous
import functools

import jax
import jax.numpy as jnp
from jax import lax
from jax.experimental import pallas as pl
from jax.experimental.pallas import tpu as pltpu
from jax.experimental.pallas import tpu_sc as plsc

MAXLEN = 200
EMBED_DIM = 32
CLASS_NUM = 128
BATCH = 4096
VOCAB = 1000000
PKW = EMBED_DIM // 2

NUM_CORES = 2
NUM_SUBCORES = 16
NUM_WORKERS = NUM_CORES * NUM_SUBCORES

HALF = 100
SAMPLES_PER_WORKER = BATCH // NUM_WORKERS
CHUNK_S = 8
CHUNK_ROWS = CHUNK_S * MAXLEN
CHUNK_GROUPS = 2 * CHUNK_S
NUM_CHUNKS = SAMPLES_PER_WORKER // CHUNK_S
GROUPS_PER_WORKER = SAMPLES_PER_WORKER * 2

_TR_C = 65536
_TR_GRID = -(-VOCAB // _TR_C)
_TR_ROWS_PAD = _TR_GRID * _TR_C
_TR_E = _TR_C // 8
_TR_E_SH = _TR_E.bit_length() - 1


def _sc_pool_body(idx_hbm, table_hbm, out_hbm, idx_v, idx2_0, idx2_1,
                  rows_0, rows_1, pooled_v, sem0, sem1):
    wid = lax.axis_index("s") * NUM_CORES + lax.axis_index("c")
    gbase_w = wid * GROUPS_PER_WORKER

    idx2 = (idx2_0, idx2_1)
    rows = (rows_0, rows_1)
    sems = (sem0, sem1)

    zero = jnp.zeros((16,), jnp.float32)
    inv = 1.0 / MAXLEN

    def stage_and_fire(c, p):
        pltpu.sync_copy(idx_hbm.at[pl.ds(gbase_w + c * CHUNK_GROUPS,
                                         CHUNK_GROUPS)], idx_v)
        for g in range(CHUNK_GROUPS):
            for j in (0, 16, 32, 48, 64, 80, HALF - 16):
                v = idx_v[g, j:j + 16]
                t = v & (_TR_C - 1)
                idx2[p][g, j:j + 16] = ((v ^ t) + ((t & (_TR_E - 1)) << 3)
                                        + (t >> _TR_E_SH))
        for g in range(CHUNK_GROUPS):
            pltpu.async_copy(table_hbm.at[idx2[p].at[g]],
                             rows[p].at[pl.ds(g * HALF, HALF)], sems[p])

    def wait_chunk(p):
        pltpu.make_async_copy(table_hbm.at[pl.ds(0, CHUNK_ROWS)],
                              rows[p], sems[p]).wait()

    def reduce_chunk(c, p):
        for s in range(CHUNK_S):
            base = s * MAXLEN

            def rbody(r, carry, base=base, rv=rows[p]):
                accs = list(carry)
                r0 = base + r * 4
                for k in range(4):
                    u = lax.bitcast_convert_type(rv[r0 + k, 0:16],
                                                 jnp.int32)
                    a = lax.bitcast_convert_type(u << 16, jnp.float32)
                    b = lax.bitcast_convert_type(u & jnp.int32(-65536),
                                                 jnp.float32)
                    accs[k] = accs[k] + a
                    accs[4 + k] = accs[4 + k] + b
                return tuple(accs)

            accs = lax.fori_loop(0, MAXLEN // 4, rbody, (zero,) * 8)
            lo = (accs[0] + accs[1]) + (accs[2] + accs[3])
            hi = (accs[4] + accs[5]) + (accs[6] + accs[7])
            row = c * CHUNK_S + s
            pooled_v[row, 0:16] = lo * inv
            pooled_v[row, 16:32] = hi * inv

    stage_and_fire(0, 0)

    def pair_body(i, _):
        for b in (0, 1):
            c = 2 * i + b
            if b == 0:
                stage_and_fire(c + 1, 1)
            else:
                @pl.when(i < NUM_CHUNKS // 2 - 1)
                def _():
                    stage_and_fire(c + 1, 0)
            wait_chunk(b)
            reduce_chunk(c, b)
        return 0

    lax.fori_loop(0, NUM_CHUNKS // 2, pair_body, 0)
    pltpu.sync_copy(pooled_v,
                    out_hbm.at[pl.ds(wid * SAMPLES_PER_WORKER,
                                     SAMPLES_PER_WORKER)])


@functools.cache
def _build_sc_pool():
    return pl.kernel(
        _sc_pool_body,
        mesh=plsc.VectorSubcoreMesh(core_axis_name="c", subcore_axis_name="s"),
        compiler_params=pltpu.CompilerParams(use_tc_tiling_on_sc=False),
        out_type=jax.ShapeDtypeStruct((BATCH, EMBED_DIM), jnp.float32),
        scratch_types=[
            pltpu.VMEM((CHUNK_GROUPS, HALF), jnp.int32),
            pltpu.VMEM((CHUNK_GROUPS, HALF), jnp.int32),
            pltpu.VMEM((CHUNK_GROUPS, HALF), jnp.int32),
            pltpu.VMEM((CHUNK_ROWS, PKW), jnp.float32),
            pltpu.VMEM((CHUNK_ROWS, PKW), jnp.float32),
            pltpu.VMEM((SAMPLES_PER_WORKER, EMBED_DIM), jnp.float32),
            pltpu.SemaphoreType.DMA,
            pltpu.SemaphoreType.DMA,
        ],
    )


def _transpose_body(in_ref, out_ref):
    row_i = lax.broadcasted_iota(jnp.int32, (8 * EMBED_DIM, 128), 0)
    lane_i = lax.broadcasted_iota(jnp.int32, (8 * EMBED_DIM, 128), 1)
    e_i = row_i // EMBED_DIM
    d_i = row_i % EMBED_DIM
    w = lane_i - PKW * e_i
    in_seg = (w >= 0) & (w < PKW)
    sel_lo = (in_seg & (d_i == 2 * w)).astype(jnp.bfloat16)
    sel_hi = (in_seg & (d_i == 2 * w + 1)).astype(jnp.bfloat16)
    x = jnp.concatenate(
        [in_ref[:, _TR_E * e:_TR_E * (e + 1)] for e in range(8)],
        axis=0).astype(jnp.bfloat16)
    acc_lo = lax.dot_general(x, sel_lo, (((0,), (0,)), ((), ())),
                             preferred_element_type=jnp.float32)
    acc_hi = lax.dot_general(x, sel_hi, (((0,), (0,)), ((), ())),
                             preferred_element_type=jnp.float32)
    u_lo = lax.bitcast_convert_type(acc_lo.astype(jnp.bfloat16),
                                    jnp.uint16).astype(jnp.uint32)
    u_hi = lax.bitcast_convert_type(acc_hi.astype(jnp.bfloat16),
                                    jnp.uint16).astype(jnp.uint32)
    out_ref[...] = lax.bitcast_convert_type(u_lo | (u_hi << 16), jnp.float32)


def _to_rowmajor_packed(table_t):
    return pl.pallas_call(
        _transpose_body,
        grid=(_TR_GRID,),
        in_specs=[pl.BlockSpec((EMBED_DIM, _TR_C), lambda i: (0, i))],
        out_specs=pl.BlockSpec((_TR_E, 128), lambda i: (i, 0)),
        out_shape=jax.ShapeDtypeStruct((_TR_ROWS_PAD * PKW // 128, 128),
                                       jnp.float32),
    )(table_t)


def _dense_softmax_body(x_ref, w_ref, b_ref, o_ref):
    logits = jnp.dot(x_ref[...], w_ref[...],
                     preferred_element_type=jnp.float32) + b_ref[...]
    m = jnp.max(logits, axis=-1, keepdims=True)
    e = jnp.exp(logits - m)
    o_ref[...] = e / jnp.sum(e, axis=-1, keepdims=True)


_TC_BLOCK = 512


def _dense_softmax(pooled, dense_w, dense_b2d):
    return pl.pallas_call(
        _dense_softmax_body,
        grid=(BATCH // _TC_BLOCK,),
        in_specs=[
            pl.BlockSpec((_TC_BLOCK, EMBED_DIM), lambda i: (i, 0)),
            pl.BlockSpec((EMBED_DIM, CLASS_NUM), lambda i: (0, 0)),
            pl.BlockSpec((1, CLASS_NUM), lambda i: (0, 0)),
        ],
        out_specs=pl.BlockSpec((_TC_BLOCK, CLASS_NUM), lambda i: (i, 0)),
        out_shape=jax.ShapeDtypeStruct((BATCH, CLASS_NUM), jnp.float32),
    )(pooled, dense_w, dense_b2d)


def kernel(inputs, embedding_table, dense_w, dense_b):
    idx = inputs.astype(jnp.int32).reshape(BATCH * 2, HALF)
    t128 = _to_rowmajor_packed(embedding_table.T)
    tbl = t128.reshape(_TR_ROWS_PAD * PKW).reshape(_TR_ROWS_PAD, PKW)
    pooled = _build_sc_pool()(idx, tbl)
    perm = jnp.array([2 * i for i in range(PKW)]
                     + [2 * i + 1 for i in range(PKW)], dtype=jnp.int32)
    w_perm = jnp.take(dense_w, perm, axis=0)
    return _dense_softmax(pooled, w_perm,
                          dense_b.reshape(1, CLASS_NUM).astype(jnp.float32))
